# baseline (speedup 1.0000x reference)
import jax
import jax.numpy as jnp
from jax.experimental import pallas as pl
from jax.experimental.pallas import tpu as pltpu

_TB = 64
_NBUF = 3
_NCOPY = 2
_CORES = 2


def _chunk_compute(x, lens, v_ref, w_ref, b_ref):
    TB, S, H = x.shape

    xr = x.reshape(TB * S, H)
    logits = jax.lax.dot_general(
        xr, v_ref[...].reshape(1, H),
        dimension_numbers=(((1,), (1,)), ((), ())),
        preferred_element_type=jnp.float32,
    ).reshape(TB, S, 1)

    m = jnp.max(logits, axis=1, keepdims=True)
    un = jnp.exp(logits - m)

    t = jax.lax.broadcasted_iota(jnp.int32, (TB, S, 1), 1)
    w_s = jnp.where(t < lens.reshape(TB, 1, 1), un, 0.0)

    denom = jnp.sum(w_s, axis=1)
    rep_un = jnp.sum(x * w_s, axis=1)
    rep = rep_un * pl.reciprocal(denom, approx=False)

    return jax.lax.dot_general(
        rep, w_ref[...],
        dimension_numbers=(((1,), (1,)), ((), ())),
        preferred_element_type=jnp.float32,
    ) + b_ref[...].reshape(1, b_ref.shape[0])


def _attn_pool_head_kernel(x_hbm, len_ref, v_ref, w_ref, b_ref, out_ref,
                           buf, lens2, sem):
    rows = out_ref.shape[0]
    n_chunks = rows // _TB
    lens2[...] = len_ref[...].reshape(rows, 1)
    base = pl.program_id(0) * rows
    part = _TB // _NCOPY

    def _copies(c, slot):
        for k in range(_NCOPY):
            yield pltpu.make_async_copy(
                x_hbm.at[pl.ds(base + c * _TB + k * part, part)],
                buf.at[slot, pl.ds(k * part, part)],
                sem.at[slot, k],
            )

    def _issue(c, slot):
        for cp in _copies(c, slot):
            cp.start()

    for c in range(min(_NBUF, n_chunks)):
        _issue(c, c)

    def _step(c, carry):
        slot = jax.lax.rem(c, _NBUF)
        for cp in _copies(c, slot):
            cp.wait()
        x = buf[slot]
        lens = lens2[pl.ds(c * _TB, _TB), :]
        out_ref[pl.ds(c * _TB, _TB), :] = _chunk_compute(
            x, lens, v_ref, w_ref, b_ref)
        nxt = c + _NBUF

        @pl.when(nxt < n_chunks)
        def _():
            _issue(nxt, slot)

        return carry

    jax.lax.fori_loop(0, n_chunks, _step, 0)


def kernel(inputs, lengths, attention_vector, weight, bias):
    B, S, H = inputs.shape
    NI = weight.shape[0]

    chunk_rows = _CORES * _TB
    B_pad = ((B + chunk_rows - 1) // chunk_rows) * chunk_rows
    rows = B_pad // _CORES

    x = inputs.astype(jnp.float32)
    lens = lengths.astype(jnp.int32)
    if B_pad != B:
        x = jnp.pad(x, ((0, B_pad - B), (0, 0), (0, 0)))
        lens = jnp.pad(lens, (0, B_pad - B), constant_values=1)
    v_1d = attention_vector.astype(jnp.float32)
    w = weight.astype(jnp.float32)
    b_1d = bias.astype(jnp.float32)

    chunk_bytes = _TB * S * H * 4
    cost = pl.CostEstimate(
        flops=int(4 * B_pad * S * H + 2 * B_pad * H * NI),
        transcendentals=int(B_pad * S),
        bytes_accessed=int(B_pad * S * H * 4 + (NI * H + NI + H) * 4
                           + B_pad * NI * 4),
    )

    out = pl.pallas_call(
        _attn_pool_head_kernel,
        out_shape=jax.ShapeDtypeStruct((B_pad, NI), jnp.float32),
        grid=(_CORES,),
        in_specs=[
            pl.BlockSpec(memory_space=pl.ANY),
            pl.BlockSpec((rows,), lambda i: (i,)),
            pl.BlockSpec((H,), lambda i: (0,)),
            pl.BlockSpec((NI, H), lambda i: (0, 0)),
            pl.BlockSpec((NI,), lambda i: (0,)),
        ],
        out_specs=pl.BlockSpec((rows, NI), lambda i: (i, 0)),
        scratch_shapes=[
            pltpu.VMEM((_NBUF, _TB, S, H), jnp.float32),
            pltpu.VMEM((rows, 1), jnp.int32),
            pltpu.SemaphoreType.DMA((_NBUF, _NCOPY)),
        ],
        compiler_params=pltpu.CompilerParams(
            dimension_semantics=("parallel",),
            vmem_limit_bytes=int(min(100 * 1024 * 1024,
                                     (_NBUF + 4) * chunk_bytes)),
        ),
        cost_estimate=cost,
    )(x, lens, v_1d, w, b_1d)

    return out[:B] if B_pad != B else out


# speedup vs baseline: 1.0160x; 1.0160x over previous
import jax
import jax.numpy as jnp
from jax.experimental import pallas as pl
from jax.experimental.pallas import tpu as pltpu

_TB = 32
_NBUF = 4
_NCOPY = 2
_CORES = 2


def _chunk_compute(x, lens, v_ref, w_ref, b_ref):
    TB, S, H = x.shape

    xr = x.reshape(TB * S, H)
    logits = jax.lax.dot_general(
        xr, v_ref[...].reshape(1, H),
        dimension_numbers=(((1,), (1,)), ((), ())),
        preferred_element_type=jnp.float32,
    ).reshape(TB, S, 1)

    m = jnp.max(logits, axis=1, keepdims=True)
    un = jnp.exp(logits - m)

    t = jax.lax.broadcasted_iota(jnp.int32, (TB, S, 1), 1)
    w_s = jnp.where(t < lens.reshape(TB, 1, 1), un, 0.0)

    denom = jnp.sum(w_s, axis=1)
    rep_un = jnp.sum(x * w_s, axis=1)
    rep = rep_un * pl.reciprocal(denom, approx=False)

    return jax.lax.dot_general(
        rep, w_ref[...],
        dimension_numbers=(((1,), (1,)), ((), ())),
        preferred_element_type=jnp.float32,
    ) + b_ref[...].reshape(1, b_ref.shape[0])


def _attn_pool_head_kernel(x_hbm, len_ref, v_ref, w_ref, b_ref, out_ref,
                           buf, lens2, sem):
    rows = out_ref.shape[0]
    n_chunks = rows // _TB
    lens2[...] = len_ref[...].reshape(rows, 1)
    base = pl.program_id(0) * rows
    part = _TB // _NCOPY

    def _copies(c, slot):
        for k in range(_NCOPY):
            yield pltpu.make_async_copy(
                x_hbm.at[pl.ds(base + c * _TB + k * part, part)],
                buf.at[slot, pl.ds(k * part, part)],
                sem.at[slot, k],
            )

    def _issue(c, slot):
        for cp in _copies(c, slot):
            cp.start()

    for c in range(min(_NBUF, n_chunks)):
        _issue(c, c)

    def _step(c, carry):
        slot = jax.lax.rem(c, _NBUF)
        for cp in _copies(c, slot):
            cp.wait()
        x = buf[slot]
        lens = lens2[pl.ds(c * _TB, _TB), :]
        out_ref[pl.ds(c * _TB, _TB), :] = _chunk_compute(
            x, lens, v_ref, w_ref, b_ref)
        nxt = c + _NBUF

        @pl.when(nxt < n_chunks)
        def _():
            _issue(nxt, slot)

        return carry

    jax.lax.fori_loop(0, n_chunks, _step, 0)


def kernel(inputs, lengths, attention_vector, weight, bias):
    B, S, H = inputs.shape
    NI = weight.shape[0]

    chunk_rows = _CORES * _TB
    B_pad = ((B + chunk_rows - 1) // chunk_rows) * chunk_rows
    rows = B_pad // _CORES

    x = inputs.astype(jnp.float32)
    lens = lengths.astype(jnp.int32)
    if B_pad != B:
        x = jnp.pad(x, ((0, B_pad - B), (0, 0), (0, 0)))
        lens = jnp.pad(lens, (0, B_pad - B), constant_values=1)
    v_1d = attention_vector.astype(jnp.float32)
    w = weight.astype(jnp.float32)
    b_1d = bias.astype(jnp.float32)

    chunk_bytes = _TB * S * H * 4
    cost = pl.CostEstimate(
        flops=int(4 * B_pad * S * H + 2 * B_pad * H * NI),
        transcendentals=int(B_pad * S),
        bytes_accessed=int(B_pad * S * H * 4 + (NI * H + NI + H) * 4
                           + B_pad * NI * 4),
    )

    out = pl.pallas_call(
        _attn_pool_head_kernel,
        out_shape=jax.ShapeDtypeStruct((B_pad, NI), jnp.float32),
        grid=(_CORES,),
        in_specs=[
            pl.BlockSpec(memory_space=pl.ANY),
            pl.BlockSpec((rows,), lambda i: (i,)),
            pl.BlockSpec((H,), lambda i: (0,)),
            pl.BlockSpec((NI, H), lambda i: (0, 0)),
            pl.BlockSpec((NI,), lambda i: (0,)),
        ],
        out_specs=pl.BlockSpec((rows, NI), lambda i: (i, 0)),
        scratch_shapes=[
            pltpu.VMEM((_NBUF, _TB, S, H), jnp.float32),
            pltpu.VMEM((rows, 1), jnp.int32),
            pltpu.SemaphoreType.DMA((_NBUF, _NCOPY)),
        ],
        compiler_params=pltpu.CompilerParams(
            dimension_semantics=("parallel",),
            vmem_limit_bytes=int(min(100 * 1024 * 1024,
                                     (_NBUF + 4) * chunk_bytes)),
        ),
        cost_estimate=cost,
    )(x, lens, v_1d, w, b_1d)

    return out[:B] if B_pad != B else out
